# Initial kernel scaffold; baseline (speedup 1.0000x reference)
import jax
import jax.numpy as jnp
from jax.experimental import pallas as pl


def kernel(x, W1, b1, W2, b2, W3, b3, D1, d1, D2, d2, D3, d3, codebook):
    raise NotImplementedError("write your pallas kernel here")


# speedup vs baseline: 1.0745x; 1.0745x over previous
import jax
import jax.numpy as jnp
from jax import lax
from jax.experimental import pallas as pl
from jax.experimental.pallas import tpu as pltpu

_B = 4
_C = 16
_H = 218
_NPOS = _H * _H
_NPOSP = 47616
_TILE = 512
_NT = _NPOSP // _TILE
_K = 8192
_NCB = 2048


def _vq_body(z_ref, cb_ref, idx_ref):
    zb = z_ref[0]
    zsq = jnp.sum(zb * zb, axis=0)

    def chunk(k, carry):
        best_d, best_i = carry
        cbc = cb_ref[pl.ds(k * _NCB, _NCB), :]
        s = lax.dot_general(cbc.astype(jnp.bfloat16), zb.astype(jnp.bfloat16),
                            (((1,), (0,)), ((), ())),
                            preferred_element_type=jnp.float32)
        cs = jnp.sum(cbc * cbc, axis=1)
        d = (zsq[None, :] - 2.0 * s) + cs[:, None]
        cmin = jnp.min(d, axis=0)
        rows = lax.broadcasted_iota(jnp.int32, (_NCB, _TILE), 0) + k * _NCB
        cidx = jnp.min(jnp.where(d == cmin[None, :], rows, jnp.int32(2**30)),
                       axis=0)
        take = cmin < best_d
        return (jnp.where(take, cmin, best_d), jnp.where(take, cidx, best_i))

    init = (jnp.full((_TILE,), jnp.inf, jnp.float32),
            jnp.zeros((_TILE,), jnp.int32))
    _, best_i = lax.fori_loop(0, _K // _NCB, chunk, init)
    idx_ref[0, 0, 0] = best_i


def _vq_argmin(z_flat, codebook):
    return pl.pallas_call(
        _vq_body,
        grid=(_B, _NT),
        in_specs=[
            pl.BlockSpec((1, _C, _TILE), lambda b, t: (b, 0, t)),
            pl.BlockSpec((_K, _C), lambda b, t: (0, 0)),
        ],
        out_specs=pl.BlockSpec((1, 1, 1, _TILE), lambda b, t: (b, t, 0, 0)),
        out_shape=jax.ShapeDtypeStruct((_B, _NT, 1, _TILE), jnp.int32),
    )(z_flat, codebook)


def _conv(x, w, b):
    y = lax.conv_general_dilated(x, w, (1, 1), 'VALID',
                                 dimension_numbers=('NCHW', 'OIHW', 'NCHW'))
    return y + b[None, :, None, None]


def _convT(x, w, b):
    y = lax.conv_transpose(x, w, (1, 1), 'VALID',
                           dimension_numbers=('NCHW', 'IOHW', 'NCHW'))
    return y + b[None, :, None, None]


def kernel(x, W1, b1, W2, b2, W3, b3, D1, d1, D2, d2, D3, d3, codebook):
    h = jax.nn.selu(_conv(x, W1, b1))
    h = jax.nn.selu(_conv(h, W2, b2))
    z = _conv(h, W3, b3)

    z_flat = jnp.pad(z.reshape(_B, _C, _NPOS),
                     ((0, 0), (0, 0), (0, _NPOSP - _NPOS)))
    idx4 = _vq_argmin(z_flat, codebook)
    idx = idx4.reshape(_B, _NPOSP)[:, :_NPOS].reshape(_B, _H, _H)

    zq = jnp.take(codebook, idx.reshape(-1), axis=0)
    zq = zq.reshape(_B, _H, _H, _C).transpose(0, 3, 1, 2)

    g = jax.nn.selu(_convT(zq, D1, d1))
    g = jax.nn.selu(_convT(g, D2, d2))
    recon = _convT(g, D3, d3)
    return recon, idx


# speedup vs baseline: 1.2922x; 1.2025x over previous
import jax
import jax.numpy as jnp
from jax import lax
from jax.experimental import pallas as pl
from jax.experimental.pallas import tpu as pltpu

_B = 4
_C = 16
_H = 218
_NPOS = _H * _H
_NPOSP = 47616
_TILE = 512
_NT = _NPOSP // _TILE
_K = 8192
_NCB = 2048


def _vq_body(z_ref, cb_ref, cbh_ref, idx_ref, cs_ref):
    @pl.when(jnp.logical_and(pl.program_id(0) == 0, pl.program_id(1) == 0))
    def _():
        cbf = cb_ref[...]
        cs_ref[...] = jnp.sum(cbf * cbf, axis=1, keepdims=True)

    zb = z_ref[0]
    zsq = jnp.sum(zb * zb, axis=0, keepdims=True)
    s2 = lax.dot_general(cbh_ref[...], zb.astype(jnp.bfloat16),
                         (((1,), (0,)), ((), ())),
                         preferred_element_type=jnp.float32)
    d = (zsq + s2) + cs_ref[...]
    cmin = jnp.min(d, axis=0, keepdims=True)
    rows = lax.broadcasted_iota(jnp.int32, (_K, _TILE), 0)
    sidx = jnp.sum(jnp.where(d == cmin, rows, 0), axis=0)
    idx_ref[0, 0, 0] = jnp.minimum(sidx, _K - 1)


def _vq_argmin(z_flat, codebook):
    return pl.pallas_call(
        _vq_body,
        grid=(_B, _NT),
        in_specs=[
            pl.BlockSpec((1, _C, _TILE), lambda b, t: (b, 0, t)),
            pl.BlockSpec((_K, _C), lambda b, t: (0, 0)),
            pl.BlockSpec((_K, _C), lambda b, t: (0, 0)),
        ],
        out_specs=pl.BlockSpec((1, 1, 1, _TILE), lambda b, t: (b, t, 0, 0)),
        out_shape=jax.ShapeDtypeStruct((_B, _NT, 1, _TILE), jnp.int32),
        scratch_shapes=[pltpu.VMEM((_K, 1), jnp.float32)],
    )(z_flat, codebook, codebook.astype(jnp.bfloat16) * jnp.bfloat16(-2.0))


def _conv(x, w, b):
    y = lax.conv_general_dilated(x, w, (1, 1), 'VALID',
                                 dimension_numbers=('NCHW', 'OIHW', 'NCHW'))
    return y + b[None, :, None, None]


def _convT(x, w, b):
    y = lax.conv_transpose(x, w, (1, 1), 'VALID',
                           dimension_numbers=('NCHW', 'IOHW', 'NCHW'))
    return y + b[None, :, None, None]


def kernel(x, W1, b1, W2, b2, W3, b3, D1, d1, D2, d2, D3, d3, codebook):
    h = jax.nn.selu(_conv(x, W1, b1))
    h = jax.nn.selu(_conv(h, W2, b2))
    z = _conv(h, W3, b3)

    z_flat = jnp.pad(z.reshape(_B, _C, _NPOS),
                     ((0, 0), (0, 0), (0, _NPOSP - _NPOS)))
    idx4 = _vq_argmin(z_flat, codebook)
    idx = idx4.reshape(_B, _NPOSP)[:, :_NPOS].reshape(_B, _H, _H)

    zq = jnp.take(codebook, idx.reshape(-1), axis=0)
    zq = zq.reshape(_B, _H, _H, _C).transpose(0, 3, 1, 2)

    g = jax.nn.selu(_convT(zq, D1, d1))
    g = jax.nn.selu(_convT(g, D2, d2))
    recon = _convT(g, D3, d3)
    return recon, idx


# speedup vs baseline: 1.3168x; 1.0190x over previous
import functools

import jax
import jax.numpy as jnp
from jax import lax
from jax.experimental import pallas as pl
from jax.experimental.pallas import tpu as pltpu
from jax.experimental.pallas import tpu_sc as plsc

_B = 4
_C = 16
_H = 218
_NPOS = _H * _H
_NPOSP = 47616
_TILE = 512
_NT = _NPOSP // _TILE
_K = 8192
_NCB = 2048


def _vq_body(z_ref, cb_ref, cbh_ref, idx_ref, cs_ref):
    @pl.when(jnp.logical_and(pl.program_id(0) == 0, pl.program_id(1) == 0))
    def _():
        cbf = cb_ref[...]
        cs_ref[...] = jnp.sum(cbf * cbf, axis=1, keepdims=True)

    zb = z_ref[0]
    zsq = jnp.sum(zb * zb, axis=0, keepdims=True)
    s2 = lax.dot_general(cbh_ref[...], zb.astype(jnp.bfloat16),
                         (((1,), (0,)), ((), ())),
                         preferred_element_type=jnp.float32)
    d = (zsq + s2) + cs_ref[...]
    cmin = jnp.min(d, axis=0, keepdims=True)
    rows = lax.broadcasted_iota(jnp.int32, (_K, _TILE), 0)
    sidx = jnp.sum(jnp.where(d == cmin, rows, 0), axis=0)
    idx_ref[0, 0, 0] = jnp.minimum(sidx, _K - 1)


def _vq_argmin(z_flat, codebook):
    return pl.pallas_call(
        _vq_body,
        grid=(_B, _NT),
        in_specs=[
            pl.BlockSpec((1, _C, _TILE), lambda b, t: (b, 0, t)),
            pl.BlockSpec((_K, _C), lambda b, t: (0, 0)),
            pl.BlockSpec((_K, _C), lambda b, t: (0, 0)),
        ],
        out_specs=pl.BlockSpec((1, 1, 1, _TILE), lambda b, t: (b, t, 0, 0)),
        out_shape=jax.ShapeDtypeStruct((_B, _NT, 1, _TILE), jnp.int32),
        scratch_shapes=[pltpu.VMEM((_K, 1), jnp.float32)],
    )(z_flat, codebook, codebook.astype(jnp.bfloat16) * jnp.bfloat16(-2.0))


_NW = 32
_WPB = _NW // _B
_GCH = 48
_PW = _GCH * 128
_NPG = _WPB * _PW


def _gather_body(cb_hbm, idx_hbm, out_hbm, idx2v, rowsv, sem):
    wid = lax.axis_index("s") * 2 + lax.axis_index("c")
    base = wid * _PW
    pltpu.sync_copy(idx_hbm.at[pl.ds(wid * _GCH, _GCH), :], idx2v)

    cps = []
    for j in range(_GCH):
        cps.append(pltpu.async_copy(
            cb_hbm.at[idx2v.at[j]], rowsv.at[pl.ds(j * 128, 128), :], sem))
    for cp in cps:
        cp.wait()

    pltpu.sync_copy(rowsv, out_hbm.at[pl.ds(base, _PW), :])


def _sc_lookup(cb, idx_pad):
    mesh = plsc.VectorSubcoreMesh(core_axis_name="c", subcore_axis_name="s")
    return pl.kernel(
        _gather_body,
        out_type=jax.ShapeDtypeStruct((_B * _NPG, _C), jnp.float32),
        mesh=mesh,
        compiler_params=pltpu.CompilerParams(use_tc_tiling_on_sc=False),
        scratch_types=[
            pltpu.VMEM((_GCH, 128), jnp.int32),
            pltpu.VMEM((_PW, _C), jnp.float32),
            pltpu.SemaphoreType.DMA,
        ],
    )(cb, idx_pad.reshape(_NW * _GCH, 128))


def _conv(x, w, b):
    y = lax.conv_general_dilated(x, w, (1, 1), 'VALID',
                                 dimension_numbers=('NCHW', 'OIHW', 'NCHW'))
    return y + b[None, :, None, None]


def _convT(x, w, b):
    y = lax.conv_transpose(x, w, (1, 1), 'VALID',
                           dimension_numbers=('NCHW', 'IOHW', 'NCHW'))
    return y + b[None, :, None, None]


def kernel(x, W1, b1, W2, b2, W3, b3, D1, d1, D2, d2, D3, d3, codebook):
    h = jax.nn.selu(_conv(x, W1, b1))
    h = jax.nn.selu(_conv(h, W2, b2))
    z = _conv(h, W3, b3)

    z_flat = jnp.pad(z.reshape(_B, _C, _NPOS),
                     ((0, 0), (0, 0), (0, _NPOSP - _NPOS)))
    idx4 = _vq_argmin(z_flat, codebook)
    idx = idx4.reshape(_B, _NPOSP)[:, :_NPOS].reshape(_B, _H, _H)

    idx_pad = jnp.pad(idx.reshape(_B, _NPOS), ((0, 0), (0, _NPG - _NPOS)))
    zq_rows = _sc_lookup(codebook, idx_pad)
    zq = (zq_rows.reshape(_B, _NPG, _C)[:, :_NPOS]
          .transpose(0, 2, 1).reshape(_B, _C, _H, _H))

    g = jax.nn.selu(_convT(zq, D1, d1))
    g = jax.nn.selu(_convT(g, D2, d2))
    recon = _convT(g, D3, d3)
    return recon, idx
